# Initial kernel scaffold; baseline (speedup 1.0000x reference)
import functools

import jax
import jax.numpy as jnp
from jax import lax
from jax.experimental import pallas as pl
from jax.experimental.pallas import tpu as pltpu
from jax.experimental.pallas import tpu_sc as plsc

B, L, N, E = 2, 4, 2500, 40000
FIN, HID, HEADS, SEQH = 2, 64, 4, 128
G = B * L
F = HEADS * HID
E2 = E + N
NCORE, NSUB = 2, 16
NCH, CH = 42, 64
EPAD = NSUB * NCH * CH
GPC = G // NCORE
ZR = 160
ZSTEP = 156


def _sc_edges_body(xl_hbm, xr_hbm, srcg_hbm, dstg_hbm, dstl_hbm, att_hbm,
                   zu_hbm, zs_hbm, U_hbm, S_hbm,
                   att_v, src_i, dstg_i, dstl_i, xl_r, xr_r, u_r, s_r,
                   U_sh, S_sh):
    c = lax.axis_index("c")
    s = lax.axis_index("s")
    pltpu.sync_copy(att_hbm, att_v)
    att_rows = [att_v[i] for i in range(F // 16)]
    lanes = lax.iota(jnp.int32, 16)
    onehot = [(lanes == h).astype(jnp.float32) for h in range(HEADS)]
    zoff = s * ZSTEP

    @pl.loop(0, GPC)
    def _graph(gi):
        g = c * GPC + gi
        pltpu.sync_copy(zu_hbm, U_sh.at[pl.ds(zoff, ZR)])
        pltpu.sync_copy(zs_hbm, S_sh.at[pl.ds(zoff, ZR)])
        pltpu.sync_copy(srcg_hbm.at[g, s], src_i)
        pltpu.sync_copy(dstg_hbm.at[g, s], dstg_i)
        pltpu.sync_copy(dstl_hbm.at[g, s], dstl_i)
        plsc.subcore_barrier()

        @pl.loop(0, NCH)
        def _chunk(j):
            pltpu.sync_copy(xl_hbm.at[src_i.at[j]], xl_r)
            pltpu.sync_copy(xr_hbm.at[dstg_i.at[j]], xr_r)
            ebase = (s * NCH + j) * CH

            @pl.loop(0, CH)
            def _edge(t):
                validf = jnp.where(ebase + t < E2, 1.0, 0.0)
                svec = jnp.zeros((16,), jnp.float32)
                for h in range(HEADS):
                    acc = None
                    a_regs = []
                    for k in range(HID // 16):
                        off = h * HID + k * 16
                        a = xl_r[t, pl.ds(off, 16)]
                        b = xr_r[t, pl.ds(off, 16)]
                        z = a + b
                        lr = jnp.maximum(z, 0.2 * z)
                        term = lr * att_rows[off // 16]
                        acc = term if acc is None else acc + term
                        a_regs.append(a)
                    e_h = jnp.sum(acc)
                    ex = jnp.exp(jnp.full((16,), e_h, jnp.float32)) * validf
                    for k in range(HID // 16):
                        off = h * HID + k * 16
                        u_r[t, pl.ds(off, 16)] = ex * a_regs[k]
                    svec = svec + ex * onehot[h]
                s_r[t] = svec

            pltpu.sync_copy(u_r, U_sh.at[dstl_i.at[j]], add=True)
            pltpu.sync_copy(s_r, S_sh.at[dstl_i.at[j]], add=True)

        plsc.subcore_barrier()
        pltpu.sync_copy(U_sh.at[pl.ds(zoff, ZR)], U_hbm.at[g, pl.ds(zoff, ZR)])
        pltpu.sync_copy(S_sh.at[pl.ds(zoff, ZR)], S_hbm.at[g, pl.ds(zoff, ZR)])
        plsc.subcore_barrier()


_sc_edges = pl.kernel(
    _sc_edges_body,
    out_type=[
        jax.ShapeDtypeStruct((G, N, F), jnp.float32),
        jax.ShapeDtypeStruct((G, N, 16), jnp.float32),
    ],
    mesh=plsc.VectorSubcoreMesh(
        core_axis_name="c", subcore_axis_name="s",
        num_cores=NCORE, num_subcores=NSUB),
    scratch_types=[
        pltpu.VMEM((F // 16, 16), jnp.float32),
        pltpu.VMEM((NCH, CH), jnp.int32),
        pltpu.VMEM((NCH, CH), jnp.int32),
        pltpu.VMEM((NCH, CH), jnp.int32),
        pltpu.VMEM((CH, F), jnp.float32),
        pltpu.VMEM((CH, F), jnp.float32),
        pltpu.VMEM((CH, F), jnp.float32),
        pltpu.VMEM((CH, 16), jnp.float32),
        pltpu.VMEM_SHARED((N, F), jnp.float32),
        pltpu.VMEM_SHARED((N, 16), jnp.float32),
    ],
)


def _dot(a, b):
    return jnp.dot(a, b, preferred_element_type=jnp.float32)


def _prep_body(x_ref, winT_ref, b_ref, wlT_ref, wrT_ref, xl_ref, xr_ref):
    h0 = jnp.maximum(_dot(x_ref[0], winT_ref[...]) + b_ref[...], 0.0)
    xl_ref[0] = _dot(h0, wlT_ref[...])
    xr_ref[0] = _dot(h0, wrT_ref[...])


def _norm_h(U, S, bg, gg, be, Bmat, Cmat):
    Sinv = 1.0 / (S + 1e-16)
    out64 = _dot(U * _dot(Sinv, Bmat), Cmat) + bg
    m = jnp.mean(out64, -1, keepdims=True)
    v = jnp.mean((out64 - m) ** 2, -1, keepdims=True)
    return jnp.maximum((out64 - m) * lax.rsqrt(v + 1e-5) * gg + be, 0.0)


def _normproj_body(U_ref, S_ref, bg_ref, g_ref, be_ref, Bmat_ref, Cmat_ref,
                   wlT_ref, wrT_ref, xl_ref, xr_ref):
    h = _norm_h(U_ref[0], S_ref[0], bg_ref[...], g_ref[...], be_ref[...],
                Bmat_ref[...], Cmat_ref[...])
    xl_ref[0] = _dot(h, wlT_ref[...])
    xr_ref[0] = _dot(h, wrT_ref[...])


def _normpool_body(U_ref, S_ref, bg_ref, g_ref, be_ref, Bmat_ref, Cmat_ref,
                   pooled_ref):
    h = _norm_h(U_ref[0], S_ref[0], bg_ref[...], g_ref[...], be_ref[...],
                Bmat_ref[...], Cmat_ref[...])
    pooled_ref[0] = jnp.mean(h, axis=0)


def _head_body(emb_ref, wih0_ref, whh0_ref, b0_ref, wih1_ref, whh1_ref,
               b1_ref, w1_ref, bh1_ref, w2_ref, bh2_ref, w3_ref, bh3_ref,
               out_ref):
    emb = emb_ref[...]
    h1 = jnp.zeros((B, SEQH), jnp.float32)
    c1 = jnp.zeros((B, SEQH), jnp.float32)
    h2 = jnp.zeros((B, SEQH), jnp.float32)
    c2 = jnp.zeros((B, SEQH), jnp.float32)

    def cell(xt, h, c, wih, whh, bb):
        z = _dot(xt, wih) + _dot(h, whh) + bb
        i = jax.nn.sigmoid(z[:, 0 * SEQH:1 * SEQH])
        f = jax.nn.sigmoid(z[:, 1 * SEQH:2 * SEQH])
        g = jnp.tanh(z[:, 2 * SEQH:3 * SEQH])
        o = jax.nn.sigmoid(z[:, 3 * SEQH:4 * SEQH])
        c = f * c + i * g
        return o * jnp.tanh(c), c

    for t in range(L):
        xt = emb[t * B:(t + 1) * B]
        h1, c1 = cell(xt, h1, c1, wih0_ref[...], whh0_ref[...], b0_ref[...])
        h2, c2 = cell(h1, h2, c2, wih1_ref[...], whh1_ref[...], b1_ref[...])

    y = jnp.maximum(_dot(h2, w1_ref[...]) + bh1_ref[...], 0.0)
    y = jnp.maximum(_dot(y, w2_ref[...]) + bh2_ref[...], 0.0)
    out_ref[...] = jax.nn.sigmoid(_dot(y, w3_ref[...]) + bh3_ref[...])


def _full(shape):
    return pl.BlockSpec(shape, lambda g: (0,) * len(shape))


def _tc_prep(x, winT, b_in, wlT, wrT):
    return pl.pallas_call(
        _prep_body,
        grid=(G,),
        in_specs=[
            pl.BlockSpec((1, N, FIN), lambda g: (g, 0, 0)),
            _full((FIN, HID)), _full((1, HID)),
            _full((HID, F)), _full((HID, F)),
        ],
        out_specs=[
            pl.BlockSpec((1, N, F), lambda g: (g, 0, 0)),
            pl.BlockSpec((1, N, F), lambda g: (g, 0, 0)),
        ],
        out_shape=[
            jax.ShapeDtypeStruct((G, N, F), jnp.float32),
            jax.ShapeDtypeStruct((G, N, F), jnp.float32),
        ],
    )(x, winT, b_in, wlT, wrT)


def _tc_normproj(U, S, bg, gg, be, Bmat, Cmat, wlT, wrT):
    return pl.pallas_call(
        _normproj_body,
        grid=(G,),
        in_specs=[
            pl.BlockSpec((1, N, F), lambda g: (g, 0, 0)),
            pl.BlockSpec((1, N, 16), lambda g: (g, 0, 0)),
            _full((1, HID)), _full((1, HID)), _full((1, HID)),
            _full((16, F)), _full((F, HID)),
            _full((HID, F)), _full((HID, F)),
        ],
        out_specs=[
            pl.BlockSpec((1, N, F), lambda g: (g, 0, 0)),
            pl.BlockSpec((1, N, F), lambda g: (g, 0, 0)),
        ],
        out_shape=[
            jax.ShapeDtypeStruct((G, N, F), jnp.float32),
            jax.ShapeDtypeStruct((G, N, F), jnp.float32),
        ],
    )(U, S, bg, gg, be, Bmat, Cmat, wlT, wrT)


def _tc_normpool(U, S, bg, gg, be, Bmat, Cmat):
    return pl.pallas_call(
        _normpool_body,
        grid=(G,),
        in_specs=[
            pl.BlockSpec((1, N, F), lambda g: (g, 0, 0)),
            pl.BlockSpec((1, N, 16), lambda g: (g, 0, 0)),
            _full((1, HID)), _full((1, HID)), _full((1, HID)),
            _full((16, F)), _full((F, HID)),
        ],
        out_specs=pl.BlockSpec((1, HID), lambda g: (g, 0)),
        out_shape=jax.ShapeDtypeStruct((G, HID), jnp.float32),
    )(U, S, bg, gg, be, Bmat, Cmat)


def _tc_head(emb, wih0T, whh0T, b0, wih1T, whh1T, b1, w1T, bh1, w2T, bh2,
             w3T, bh3):
    return pl.pallas_call(
        _head_body,
        in_specs=[
            _full((L * B, HID)),
            _full((HID, 4 * SEQH)), _full((SEQH, 4 * SEQH)),
            _full((1, 4 * SEQH)),
            _full((SEQH, 4 * SEQH)), _full((SEQH, 4 * SEQH)),
            _full((1, 4 * SEQH)),
            _full((SEQH, SEQH // 2)), _full((1, SEQH // 2)),
            _full((SEQH // 2, SEQH // 4)), _full((1, SEQH // 4)),
            _full((SEQH // 4, 1)), _full((1, 1)),
        ],
        out_specs=_full((B, 1)),
        out_shape=jax.ShapeDtypeStruct((B, 1), jnp.float32),
    )(emb, wih0T, whh0T, b0, wih1T, whh1T, b1, w1T, bh1, w2T, bh2, w3T, bh3)


def kernel(x, edge_index, W_in, b_in, Wl0, Wr0, att0, bg0, g0, be0,
           Wl1, Wr1, att1, bg1, g1, be1,
           Wih0, Whh0, bih0, bhh0, Wih1, Whh1, bih1, bhh1,
           Wh1, bh1, Wh2, bh2, Wh3, bh3):
    f32 = jnp.float32

    eis = edge_index.reshape(G, 2, E)
    loop = jnp.arange(N, dtype=jnp.int32)
    src = jnp.concatenate(
        [eis[:, 0, :], jnp.broadcast_to(loop, (G, N))], axis=1)
    dst = jnp.concatenate(
        [eis[:, 1, :], jnp.broadcast_to(loop, (G, N))], axis=1)
    pad = jnp.zeros((G, EPAD - E2), jnp.int32)
    srcp = jnp.concatenate([src, pad], axis=1)
    dstp = jnp.concatenate([dst, pad], axis=1)
    goff = (jnp.arange(G, dtype=jnp.int32) * N)[:, None]
    shape4 = (G, NSUB, NCH, CH)
    srcg = (srcp + goff).reshape(shape4)
    dstg = (dstp + goff).reshape(shape4)
    dstl = dstp.reshape(shape4)

    zu = jnp.zeros((ZR, F), f32)
    zs = jnp.zeros((ZR, 16), f32)

    jidx = jnp.arange(F)
    Bmat = jnp.zeros((16, F), f32).at[jidx // HID, jidx].set(1.0)
    Cmat = jnp.zeros((F, HID), f32).at[jidx, jidx % HID].set(1.0 / HEADS)

    r1 = lambda a: a.reshape(1, -1)

    xl0, xr0 = _tc_prep(x.reshape(G, N, FIN), W_in.T, r1(b_in), Wl0.T, Wr0.T)

    U0, S0 = _sc_edges(xl0.reshape(G * N, F), xr0.reshape(G * N, F),
                       srcg, dstg, dstl, att0.reshape(F // 16, 16), zu, zs)

    xl1, xr1 = _tc_normproj(U0, S0, r1(bg0), r1(g0), r1(be0), Bmat, Cmat,
                            Wl1.T, Wr1.T)

    U1, S1 = _sc_edges(xl1.reshape(G * N, F), xr1.reshape(G * N, F),
                       srcg, dstg, dstl, att1.reshape(F // 16, 16), zu, zs)

    pooled = _tc_normpool(U1, S1, r1(bg1), r1(g1), r1(be1), Bmat, Cmat)

    emb = pooled.reshape(B, L, HID).transpose(1, 0, 2).reshape(L * B, HID)
    out = _tc_head(emb, Wih0.T, Whh0.T, r1(bih0 + bhh0),
                   Wih1.T, Whh1.T, r1(bih1 + bhh1),
                   Wh1.T, r1(bh1), Wh2.T, r1(bh2), Wh3.T, r1(bh3))
    return out


# speedup vs baseline: 22.9876x; 22.9876x over previous
import functools

import jax
import jax.numpy as jnp
from jax import lax
from jax.experimental import pallas as pl
from jax.experimental.pallas import tpu as pltpu
from jax.experimental.pallas import tpu_sc as plsc

B, L, N, E = 2, 4, 2500, 40000
FIN, HID, HEADS, SEQH = 2, 64, 4, 128
G = B * L
F = HEADS * HID
E2 = E + N
NCORE, NSUB = 2, 16
CH = 64
EPAD = 43008
GPC = G // NCORE
NPAD = 2560


NCHT = G * EPAD // (NCORE * NSUB * CH)
F2 = F + 128


NCHP = NCHT + 8


def _sc_edges_body(xl_hbm, xr_hbm, srcg_hbm, dstg_hbm, att_hbm,
                   zu_hbm, U_hbm,
                   att_v, src_i, dst_i, xl_a, xr_a, xl_b, xr_b, u_r,
                   sga, sgb, ss):
    c = lax.axis_index("c")
    s = lax.axis_index("s")
    pltpu.sync_copy(att_hbm, att_v)
    att_rows = [att_v[i] for i in range(F // 16)]
    lanes = lax.iota(jnp.int32, 16)
    onehot = [(lanes == h).astype(jnp.float32) for h in range(HEADS)]

    zrows = GPC * NPAD // NSUB
    zoff = c * GPC * NPAD + s * zrows
    pltpu.sync_copy(zu_hbm, U_hbm.at[pl.ds(zoff, zrows)])
    pltpu.sync_copy(srcg_hbm.at[c, s], src_i)
    pltpu.sync_copy(dstg_hbm.at[c, s], dst_i)
    @pl.loop(0, CH)
    def _clr(t):
        for q in range(F + 16, F2, 16):
            u_r[t, pl.ds(q, 16)] = jnp.zeros((16,), jnp.float32)
    plsc.subcore_barrier()

    def gather(j, xl_r, xr_r, sem):
        pltpu.async_copy(xl_hbm.at[src_i.at[j, pl.ds(0, CH)]], xl_r, sem)
        pltpu.async_copy(xr_hbm.at[dst_i.at[j, pl.ds(0, CH)]], xr_r, sem)

    def drain_gather(j, xl_r, xr_r, sem):
        pltpu.make_async_copy(
            xl_hbm.at[src_i.at[j, pl.ds(0, CH)]], xl_r, sem).wait()
        pltpu.make_async_copy(
            xr_hbm.at[dst_i.at[j, pl.ds(0, CH)]], xr_r, sem).wait()

    def compute(xl_r, xr_r):
        @plsc.parallel_loop(0, CH, unroll=4)
        def _edge(t):
            svec = jnp.zeros((16,), jnp.float32)
            for h in range(HEADS):
                acc = None
                a_regs = []
                for q in (2 * h, 2 * h + 1):
                    va = plsc.bitcast(xl_r[t, pl.ds(16 * q, 16)],
                                      jnp.bfloat16)
                    vb = plsc.bitcast(xr_r[t, pl.ds(16 * q, 16)],
                                      jnp.bfloat16)
                    a0, a1 = plsc.unpack(
                        va, format=plsc.PackFormat.INTERLEAVED)
                    b0, b1 = plsc.unpack(
                        vb, format=plsc.PackFormat.INTERLEAVED)
                    for aa, bb, sg in ((a0, b0, 2 * q), (a1, b1, 2 * q + 1)):
                        z = aa + bb
                        lr = jnp.maximum(z, 0.2 * z)
                        term = lr * att_rows[sg]
                        acc = term if acc is None else acc + term
                        a_regs.append((aa, sg))
                e_h = jnp.sum(acc)
                ex = jnp.exp(jnp.full((16,), e_h, jnp.float32))
                for aa, sg in a_regs:
                    u_r[t, pl.ds(16 * sg, 16)] = ex * aa
                svec = svec + ex * onehot[h]
            u_r[t, pl.ds(F, 16)] = svec

    def drain_scatter():
        pltpu.make_async_copy(
            u_r, U_hbm.at[dst_i.at[0, pl.ds(0, CH)]], ss).wait()

    gather(0, xl_a, xr_a, sga)

    @pl.loop(0, NCHT, step=2)
    def _chunk(j):
        gather(j + 1, xl_b, xr_b, sgb)
        drain_gather(j, xl_a, xr_a, sga)

        @pl.when(j > 0)
        def _():
            drain_scatter()

        compute(xl_a, xr_a)
        pltpu.async_copy(
            u_r, U_hbm.at[dst_i.at[j, pl.ds(0, CH)]], ss, add=True)
        gather(j + 2, xl_a, xr_a, sga)
        drain_gather(j + 1, xl_b, xr_b, sgb)
        drain_scatter()
        compute(xl_b, xr_b)
        pltpu.async_copy(
            u_r, U_hbm.at[dst_i.at[j + 1, pl.ds(0, CH)]], ss, add=True)

    drain_gather(NCHT, xl_a, xr_a, sga)
    drain_scatter()


@functools.cache
def _build_sc_edges():
  return pl.kernel(
    _sc_edges_body,
    out_type=jax.ShapeDtypeStruct((G * NPAD, F2), jnp.float32),
    mesh=plsc.VectorSubcoreMesh(
        core_axis_name="c", subcore_axis_name="s",
        num_cores=NCORE, num_subcores=NSUB),
    compiler_params=pltpu.CompilerParams(needs_layout_passes=False),
    scratch_types=[
        pltpu.VMEM((F // 16, 16), jnp.float32),
        pltpu.VMEM((NCHP, 128), jnp.int32),
        pltpu.VMEM((NCHP, 128), jnp.int32),
        pltpu.VMEM((CH, F // 2), jnp.int32),
        pltpu.VMEM((CH, F // 2), jnp.int32),
        pltpu.VMEM((CH, F // 2), jnp.int32),
        pltpu.VMEM((CH, F // 2), jnp.int32),
        pltpu.VMEM((CH, F2), jnp.float32),
        pltpu.SemaphoreType.DMA,
        pltpu.SemaphoreType.DMA,
        pltpu.SemaphoreType.DMA,
    ],
  )


def _sc_edges(*args):
    return _build_sc_edges()(*args)


def _dot(a, b):
    return jnp.dot(a, b, preferred_element_type=jnp.float32)


def _prep_body(x_ref, winT_ref, b_ref, wlT_ref, wrT_ref, xl_ref, xr_ref):
    h0 = jnp.maximum(_dot(x_ref[0], winT_ref[...]) + b_ref[...], 0.0)
    xl_ref[0] = _dot(h0, wlT_ref[...]).astype(jnp.bfloat16)
    xr_ref[0] = _dot(h0, wrT_ref[...]).astype(jnp.bfloat16)


def _norm_h(Uw, bg, gg, be, Bmat, Cmat):
    U = Uw[:, :F]
    S = Uw[:, F:F + 16]
    Sinv = 1.0 / (S + 1e-16)
    out64 = _dot(U * _dot(Sinv, Bmat), Cmat) + bg
    m = jnp.mean(out64, -1, keepdims=True)
    v = jnp.mean((out64 - m) ** 2, -1, keepdims=True)
    return jnp.maximum((out64 - m) * lax.rsqrt(v + 1e-5) * gg + be, 0.0)


def _normproj_body(U_ref, bg_ref, g_ref, be_ref, Bmat_ref, Cmat_ref,
                   wlT_ref, wrT_ref, xl_ref, xr_ref):
    h = _norm_h(U_ref[0], bg_ref[...], g_ref[...], be_ref[...],
                Bmat_ref[...], Cmat_ref[...])
    xl_ref[0] = _dot(h, wlT_ref[...]).astype(jnp.bfloat16)
    xr_ref[0] = _dot(h, wrT_ref[...]).astype(jnp.bfloat16)


def _normpool_body(U_ref, bg_ref, g_ref, be_ref, Bmat_ref, Cmat_ref,
                   pooled_ref):
    h = _norm_h(U_ref[0], bg_ref[...], g_ref[...], be_ref[...],
                Bmat_ref[...], Cmat_ref[...])
    pooled_ref[0, 0] = jnp.mean(h[:N], axis=0)


def _head_body(emb_ref, wih0_ref, whh0_ref, b0_ref, wih1_ref, whh1_ref,
               b1_ref, w1_ref, bh1_ref, w2_ref, bh2_ref, w3_ref, bh3_ref,
               out_ref):
    emb = emb_ref[...]
    h1 = jnp.zeros((B, SEQH), jnp.float32)
    c1 = jnp.zeros((B, SEQH), jnp.float32)
    h2 = jnp.zeros((B, SEQH), jnp.float32)
    c2 = jnp.zeros((B, SEQH), jnp.float32)

    def cell(xt, h, c, wih, whh, bb):
        z = _dot(xt, wih) + _dot(h, whh) + bb
        i = jax.nn.sigmoid(z[:, 0 * SEQH:1 * SEQH])
        f = jax.nn.sigmoid(z[:, 1 * SEQH:2 * SEQH])
        g = jnp.tanh(z[:, 2 * SEQH:3 * SEQH])
        o = jax.nn.sigmoid(z[:, 3 * SEQH:4 * SEQH])
        c = f * c + i * g
        return o * jnp.tanh(c), c

    for t in range(L):
        xt = emb[t * B:(t + 1) * B]
        h1, c1 = cell(xt, h1, c1, wih0_ref[...], whh0_ref[...], b0_ref[...])
        h2, c2 = cell(h1, h2, c2, wih1_ref[...], whh1_ref[...], b1_ref[...])

    y = jnp.maximum(_dot(h2, w1_ref[...]) + bh1_ref[...], 0.0)
    y = jnp.maximum(_dot(y, w2_ref[...]) + bh2_ref[...], 0.0)
    out_ref[...] = jax.nn.sigmoid(_dot(y, w3_ref[...]) + bh3_ref[...])


def _full(shape):
    return pl.BlockSpec(shape, lambda *a: (0,) * len(shape))


def _tc_prep(x, winT, b_in, wlT, wrT):
    return pl.pallas_call(
        _prep_body,
        grid=(G,),
        in_specs=[
            pl.BlockSpec((1, NPAD, FIN), lambda g: (g, 0, 0)),
            _full((FIN, HID)), _full((1, HID)),
            _full((HID, F)), _full((HID, F)),
        ],
        out_specs=[
            pl.BlockSpec((1, NPAD, F), lambda g: (g, 0, 0)),
            pl.BlockSpec((1, NPAD, F), lambda g: (g, 0, 0)),
        ],
        out_shape=[
            jax.ShapeDtypeStruct((G, NPAD, F), jnp.bfloat16),
            jax.ShapeDtypeStruct((G, NPAD, F), jnp.bfloat16),
        ],
    )(x, winT, b_in, wlT, wrT)


def _tc_normproj(U, bg, gg, be, Bmat, Cmat, wlT, wrT):
    return pl.pallas_call(
        _normproj_body,
        grid=(G,),
        in_specs=[
            pl.BlockSpec((1, NPAD, F2), lambda g: (g, 0, 0)),
            _full((1, HID)), _full((1, HID)), _full((1, HID)),
            _full((16, F)), _full((F, HID)),
            _full((HID, F)), _full((HID, F)),
        ],
        out_specs=[
            pl.BlockSpec((1, NPAD, F), lambda g: (g, 0, 0)),
            pl.BlockSpec((1, NPAD, F), lambda g: (g, 0, 0)),
        ],
        out_shape=[
            jax.ShapeDtypeStruct((G, NPAD, F), jnp.bfloat16),
            jax.ShapeDtypeStruct((G, NPAD, F), jnp.bfloat16),
        ],
    )(U, bg, gg, be, Bmat, Cmat, wlT, wrT)


def _tc_normpool(U, bg, gg, be, Bmat, Cmat):
    return pl.pallas_call(
        _normpool_body,
        grid=(G,),
        in_specs=[
            pl.BlockSpec((1, NPAD, F2), lambda g: (g, 0, 0)),
            _full((1, HID)), _full((1, HID)), _full((1, HID)),
            _full((16, F)), _full((F, HID)),
        ],
        out_specs=pl.BlockSpec((1, 1, HID), lambda g: (g, 0, 0)),
        out_shape=jax.ShapeDtypeStruct((G, 1, HID), jnp.float32),
    )(U, bg, gg, be, Bmat, Cmat)


def _tc_head(emb, wih0T, whh0T, b0, wih1T, whh1T, b1, w1T, bh1, w2T, bh2,
             w3T, bh3):
    return pl.pallas_call(
        _head_body,
        in_specs=[
            _full((L * B, HID)),
            _full((HID, 4 * SEQH)), _full((SEQH, 4 * SEQH)),
            _full((1, 4 * SEQH)),
            _full((SEQH, 4 * SEQH)), _full((SEQH, 4 * SEQH)),
            _full((1, 4 * SEQH)),
            _full((SEQH, SEQH // 2)), _full((1, SEQH // 2)),
            _full((SEQH // 2, SEQH // 4)), _full((1, SEQH // 4)),
            _full((SEQH // 4, 1)), _full((1, 1)),
        ],
        out_specs=_full((B, 1)),
        out_shape=jax.ShapeDtypeStruct((B, 1), jnp.float32),
    )(emb, wih0T, whh0T, b0, wih1T, whh1T, b1, w1T, bh1, w2T, bh2, w3T, bh3)


def kernel(x, edge_index, W_in, b_in, Wl0, Wr0, att0, bg0, g0, be0,
           Wl1, Wr1, att1, bg1, g1, be1,
           Wih0, Whh0, bih0, bhh0, Wih1, Whh1, bih1, bhh1,
           Wh1, bh1, Wh2, bh2, Wh3, bh3):
    f32 = jnp.float32

    eis = edge_index.reshape(G, 2, E)
    loop = jnp.arange(N, dtype=jnp.int32)
    src = jnp.concatenate(
        [eis[:, 0, :], jnp.broadcast_to(loop, (G, N))], axis=1)
    dst = jnp.concatenate(
        [eis[:, 1, :], jnp.broadcast_to(loop, (G, N))], axis=1)
    pad = jnp.full((G, EPAD - E2), N, jnp.int32)
    srcp = jnp.concatenate([src, pad], axis=1)
    dstp = jnp.concatenate([dst, pad], axis=1)
    goff = (jnp.arange(G, dtype=jnp.int32) * NPAD)[:, None]
    shape4 = (NCORE, NSUB, NCHT, CH)
    srcg = (srcp + goff).reshape(shape4)
    dstg = (dstp + goff).reshape(shape4)
    padv = (jnp.arange(NCORE, dtype=jnp.int32) * (GPC * NPAD) + N)

    def _pad_idx(a):
        rows = jnp.broadcast_to(padv[:, None, None, None],
                                (NCORE, NSUB, NCHP - NCHT, CH))
        a = jnp.concatenate([a, rows], axis=2)
        lanes = jnp.broadcast_to(padv[:, None, None, None],
                                 (NCORE, NSUB, NCHP, 128 - CH))
        return jnp.concatenate([a, lanes], axis=3)

    srcg = _pad_idx(srcg)
    dstg = _pad_idx(dstg)

    zrows = GPC * NPAD // NSUB
    zu = jnp.zeros((zrows, F2), f32)

    sidx = jnp.arange(F)
    sg = sidx // 16
    permF = 32 * (sg // 2) + 2 * (sidx % 16) + (sg % 2)
    Bmat = (permF[None, :] // HID
            == jnp.arange(16)[:, None]).astype(f32)
    Cmat = jnp.zeros((F, HID), f32).at[sidx, permF % HID].set(1.0 / HEADS)

    def att_slots(att):
        flat = att.reshape(F)
        i16 = jnp.arange(16)
        rows = []
        for q in range(8):
            rows.append(flat[32 * q + 2 * i16])
            rows.append(flat[32 * q + 2 * i16 + 1])
        return jnp.stack(rows)

    r1 = lambda a: a.reshape(1, -1)

    xpad = jnp.concatenate(
        [x.reshape(G, N, FIN), jnp.zeros((G, NPAD - N, FIN), f32)], axis=1)
    xl0, xr0 = _tc_prep(xpad, W_in.T, r1(b_in), Wl0.T, Wr0.T)

    as32 = lambda a: lax.bitcast_convert_type(
        a.reshape(G * NPAD, F // 2, 2), jnp.int32)
    U0 = _sc_edges(as32(xl0), as32(xr0), srcg, dstg, att_slots(att0), zu)
    U0 = U0.reshape(G, NPAD, F2)

    xl1, xr1 = _tc_normproj(U0, r1(bg0), r1(g0), r1(be0), Bmat, Cmat,
                            Wl1.T, Wr1.T)

    U1 = _sc_edges(as32(xl1), as32(xr1), srcg, dstg, att_slots(att1), zu)
    U1 = U1.reshape(G, NPAD, F2)

    pooled = _tc_normpool(U1, r1(bg1), r1(g1), r1(be1), Bmat, Cmat)

    emb = pooled.reshape(B, L, HID).transpose(1, 0, 2).reshape(L * B, HID)
    out = _tc_head(emb, Wih0.T, Whh0.T, r1(bih0 + bhh0),
                   Wih1.T, Whh1.T, r1(bih1 + bhh1),
                   Wh1.T, r1(bh1), Wh2.T, r1(bh2), Wh3.T, r1(bh3))
    return out
